# initial kernel scaffold (unmeasured)
import jax
import jax.numpy as jnp
from jax import lax
from jax.experimental import pallas as pl
from jax.experimental.pallas import tpu as pltpu

N_DEV = 8
SQ = 256
SKV = 4096
HQ = 8
DH = 128
D = HQ * DH
SCALE = 0.08838834764831843
ROWS = 264


def kernel(x, Wq, K_ext, V_ext, Wo):
    def body(x_ref, wq_ref, k_ref, v_ref, wo_ref, out_ref,
             comm_ref, send_sems, recv_sems):
        my = lax.axis_index("i")
        left = lax.rem(my + N_DEV - 1, N_DEV)
        right = lax.rem(my + 1, N_DEV)

        q = jnp.dot(x_ref[0], wq_ref[...],
                    preferred_element_type=jnp.float32) * SCALE

        qb = lax.broadcasted_iota(jnp.int32, (SQ, SKV), 0) // 64
        kb = lax.broadcasted_iota(jnp.int32, (SQ, SKV), 1) // 64
        mask = (kb % 4) == (qb % 4)

        k = k_ref[0]
        v = v_ref[0]
        l_cols = []
        for h in range(HQ):
            qh = q[:, h * DH:(h + 1) * DH]
            kh = k[:, h, :]
            vh = v[:, h, :]
            s = lax.dot_general(qh, kh, (((1,), (1,)), ((), ())),
                                preferred_element_type=jnp.float32)
            w = jnp.where(mask, jnp.exp(s), 0.0)
            l_cols.append(jnp.sum(w, axis=1))
            comm_ref[0, 0:SQ, h * DH:(h + 1) * DH] = jnp.dot(
                w, vh, preferred_element_type=jnp.float32)
        l_part = jnp.stack(l_cols, axis=1)
        comm_ref[0, SQ:SQ + 2, :] = l_part.reshape(2, D)
        comm_ref[0, SQ + 2:ROWS, :] = jnp.zeros((ROWS - SQ - 2, D),
                                                jnp.float32)

        barrier_sem = pltpu.get_barrier_semaphore()
        for nbr in (left, right):
            pl.semaphore_signal(barrier_sem, inc=1, device_id=(nbr,),
                                device_id_type=pl.DeviceIdType.MESH)
        pl.semaphore_wait(barrier_sem, 2)

        for h in range(N_DEV - 1):
            rdma = pltpu.make_async_remote_copy(
                src_ref=comm_ref.at[h],
                dst_ref=comm_ref.at[h + 1],
                send_sem=send_sems.at[h],
                recv_sem=recv_sems.at[h + 1],
                device_id=(right,),
                device_id_type=pl.DeviceIdType.MESH,
            )
            rdma.start()
            rdma.wait()

        tot = jnp.sum(comm_ref[...], axis=0)
        o_sum = tot[0:SQ, :]
        l_sum = tot[SQ:SQ + 2, :].reshape(SQ, HQ)
        ctx = o_sum.reshape(SQ, HQ, DH) / l_sum[:, :, None]
        out_ref[0] = jnp.dot(ctx.reshape(SQ, D), wo_ref[...],
                             preferred_element_type=jnp.float32)

    return pl.pallas_call(
        body,
        out_shape=jax.ShapeDtypeStruct((1, SQ, D), jnp.float32),
        in_specs=[pl.BlockSpec(memory_space=pltpu.VMEM)] * 5,
        out_specs=pl.BlockSpec(memory_space=pltpu.VMEM),
        scratch_shapes=[
            pltpu.VMEM((N_DEV, ROWS, D), jnp.float32),
            pltpu.SemaphoreType.DMA((N_DEV,)),
            pltpu.SemaphoreType.DMA((N_DEV,)),
        ],
        compiler_params=pltpu.CompilerParams(collective_id=0),
    )(x, Wq, K_ext, V_ext, Wo)


# baseline (device time: 154084 ns/iter reference)
import jax
import jax.numpy as jnp
from jax import lax
from jax.experimental import pallas as pl
from jax.experimental.pallas import tpu as pltpu

N_DEV = 8
SQ = 256
SKV = 4096
HQ = 8
DH = 128
D = HQ * DH
SCALE = 0.08838834764831843
COLS = D + DH


def kernel(x, Wq, K_ext, V_ext, Wo):
    def body(x_ref, wq_ref, k_ref, v_ref, wo_ref, out_ref,
             comm_ref, send_sems, recv_sems):
        my = lax.axis_index("i")
        left = lax.rem(my + N_DEV - 1, N_DEV)
        right = lax.rem(my + 1, N_DEV)

        q = jnp.dot(x_ref[0], wq_ref[...],
                    preferred_element_type=jnp.float32) * SCALE

        qb = lax.broadcasted_iota(jnp.int32, (SQ, SKV), 0) // 64
        kb = lax.broadcasted_iota(jnp.int32, (SQ, SKV), 1) // 64
        mask = (kb % 4) == (qb % 4)

        l_cols = []
        for h in range(HQ):
            qh = q[:, h * DH:(h + 1) * DH]
            kh = k_ref[0, :, h, :]
            vh = v_ref[0, :, h, :]
            s = lax.dot_general(qh, kh, (((1,), (1,)), ((), ())),
                                preferred_element_type=jnp.float32)
            w = jnp.where(mask, jnp.exp(s), 0.0)
            l_cols.append(jnp.sum(w, axis=1, keepdims=True))
            comm_ref[0, :, h * DH:(h + 1) * DH] = jnp.dot(
                w, vh, preferred_element_type=jnp.float32)
        comm_ref[0, :, D:D + HQ] = jnp.concatenate(l_cols, axis=1)
        comm_ref[0, :, D + HQ:COLS] = jnp.zeros((SQ, COLS - D - HQ),
                                                jnp.float32)

        barrier_sem = pltpu.get_barrier_semaphore()
        for nbr in (left, right):
            pl.semaphore_signal(barrier_sem, inc=1, device_id=(nbr,),
                                device_id_type=pl.DeviceIdType.MESH)
        pl.semaphore_wait(barrier_sem, 2)

        for h in range(N_DEV - 1):
            rdma = pltpu.make_async_remote_copy(
                src_ref=comm_ref.at[h],
                dst_ref=comm_ref.at[h + 1],
                send_sem=send_sems.at[h],
                recv_sem=recv_sems.at[h + 1],
                device_id=(right,),
                device_id_type=pl.DeviceIdType.MESH,
            )
            rdma.start()
            rdma.wait()

        tot = jnp.sum(comm_ref[...], axis=0)
        ctx_cols = []
        for h in range(HQ):
            ctx_cols.append(tot[:, h * DH:(h + 1) * DH]
                            / tot[:, D + h:D + h + 1])
        ctx = jnp.concatenate(ctx_cols, axis=1)
        out_ref[0] = jnp.dot(ctx, wo_ref[...],
                             preferred_element_type=jnp.float32)

    return pl.pallas_call(
        body,
        out_shape=jax.ShapeDtypeStruct((1, SQ, D), jnp.float32),
        in_specs=[pl.BlockSpec(memory_space=pltpu.VMEM)] * 5,
        out_specs=pl.BlockSpec(memory_space=pltpu.VMEM),
        scratch_shapes=[
            pltpu.VMEM((N_DEV, SQ, COLS), jnp.float32),
            pltpu.SemaphoreType.DMA((N_DEV,)),
            pltpu.SemaphoreType.DMA((N_DEV,)),
        ],
        compiler_params=pltpu.CompilerParams(
            collective_id=0, vmem_limit_bytes=100 * 1024 * 1024),
    )(x, Wq, K_ext, V_ext, Wo)


# device time: 81004 ns/iter; 1.9022x vs baseline; 1.9022x over previous
import jax
import jax.numpy as jnp
from jax import lax
from jax.experimental import pallas as pl
from jax.experimental.pallas import tpu as pltpu

N_DEV = 8
SQ = 256
SKV = 4096
HQ = 8
DH = 128
D = HQ * DH
SCALE = 0.08838834764831843
COLS = D + DH
OWN = SQ // N_DEV


def kernel(x, Wq, K_ext, V_ext, Wo):
    def body(x_ref, wq_ref, k_ref, v_ref, wo_ref, out_ref,
             part_ref, rs0_ref, rs1_ref, rs2_ref,
             rs_send_sems, rs_recv_sems, ag_send_sems, ag_recv_sems):
        my = lax.axis_index("i")

        q = jnp.dot(x_ref[0], wq_ref[...],
                    preferred_element_type=jnp.float32) * SCALE

        qb = lax.broadcasted_iota(jnp.int32, (SQ, SKV), 0) // 64
        kb = lax.broadcasted_iota(jnp.int32, (SQ, SKV), 1) // 64
        mask = (kb % 4) == (qb % 4)

        l_cols = []
        for h in range(HQ):
            qh = q[:, h * DH:(h + 1) * DH]
            kh = k_ref[0, :, h, :]
            vh = v_ref[0, :, h, :]
            s = lax.dot_general(qh, kh, (((1,), (1,)), ((), ())),
                                preferred_element_type=jnp.float32)
            w = jnp.where(mask, jnp.exp(s), 0.0)
            l_cols.append(jnp.sum(w, axis=1, keepdims=True))
            part_ref[:, h * DH:(h + 1) * DH] = jnp.dot(
                w, vh, preferred_element_type=jnp.float32)
        part_ref[:, D:D + HQ] = jnp.concatenate(l_cols, axis=1)
        part_ref[:, D + HQ:COLS] = jnp.zeros((SQ, COLS - D - HQ),
                                             jnp.float32)

        barrier_sem = pltpu.get_barrier_semaphore()
        for m in (1, 2, 4):
            pl.semaphore_signal(barrier_sem, inc=1,
                                device_id=(lax.bitwise_xor(my, m),),
                                device_id_type=pl.DeviceIdType.MESH)
        pl.semaphore_wait(barrier_sem, 3)

        base = my * 0
        rs_recvs = (rs0_ref, rs1_ref, rs2_ref)
        for j, m in enumerate((4, 2, 1)):
            sz = SQ >> (j + 1)
            bit = lax.bitwise_and(lax.shift_right_logical(my, 2 - j), 1)
            keep_off = pl.multiple_of(base + bit * sz, 32)
            send_off = pl.multiple_of(base + (1 - bit) * sz, 32)
            rdma = pltpu.make_async_remote_copy(
                src_ref=part_ref.at[pl.ds(send_off, sz), :],
                dst_ref=rs_recvs[j],
                send_sem=rs_send_sems.at[j],
                recv_sem=rs_recv_sems.at[j],
                device_id=(lax.bitwise_xor(my, m),),
                device_id_type=pl.DeviceIdType.MESH,
            )
            rdma.start()
            rdma.wait()
            part_ref[pl.ds(keep_off, sz), :] = (
                part_ref[pl.ds(keep_off, sz), :] + rs_recvs[j][...])
            base = keep_off
        base = pl.multiple_of(base, 32)

        mine = part_ref[pl.ds(base, OWN), :]
        ctx_cols = []
        for h in range(HQ):
            ctx_cols.append(mine[:, h * DH:(h + 1) * DH]
                            / mine[:, D + h:D + h + 1])
        ctx = jnp.concatenate(ctx_cols, axis=1)
        out_ref[0, pl.ds(base, OWN), :] = jnp.dot(
            ctx, wo_ref[...], preferred_element_type=jnp.float32)

        own_off = base
        for j, m in enumerate((1, 2, 4)):
            sz = OWN << j
            own_off = pl.multiple_of(own_off, 32)
            rdma = pltpu.make_async_remote_copy(
                src_ref=out_ref.at[0, pl.ds(own_off, sz), :],
                dst_ref=out_ref.at[0, pl.ds(own_off, sz), :],
                send_sem=ag_send_sems.at[j],
                recv_sem=ag_recv_sems.at[j],
                device_id=(lax.bitwise_xor(my, m),),
                device_id_type=pl.DeviceIdType.MESH,
            )
            rdma.start()
            rdma.wait()
            own_off = own_off & ~sz

    return pl.pallas_call(
        body,
        out_shape=jax.ShapeDtypeStruct((1, SQ, D), jnp.float32),
        in_specs=[pl.BlockSpec(memory_space=pltpu.VMEM)] * 5,
        out_specs=pl.BlockSpec(memory_space=pltpu.VMEM),
        scratch_shapes=[
            pltpu.VMEM((SQ, COLS), jnp.float32),
            pltpu.VMEM((SQ // 2, COLS), jnp.float32),
            pltpu.VMEM((SQ // 4, COLS), jnp.float32),
            pltpu.VMEM((SQ // 8, COLS), jnp.float32),
            pltpu.SemaphoreType.DMA((3,)),
            pltpu.SemaphoreType.DMA((3,)),
            pltpu.SemaphoreType.DMA((3,)),
            pltpu.SemaphoreType.DMA((3,)),
        ],
        compiler_params=pltpu.CompilerParams(
            collective_id=0, vmem_limit_bytes=100 * 1024 * 1024),
    )(x, Wq, K_ext, V_ext, Wo)


# device time: 74245 ns/iter; 2.0753x vs baseline; 1.0910x over previous
import jax
import jax.numpy as jnp
from jax import lax
from jax.experimental import pallas as pl
from jax.experimental.pallas import tpu as pltpu

N_DEV = 8
SQ = 256
SKV = 4096
HQ = 8
DH = 128
D = HQ * DH
SCALE = 0.08838834764831843
COLS = D + DH
OWN = SQ // N_DEV


def kernel(x, Wq, K_ext, V_ext, Wo):
    def body(x_ref, wq_ref, k_ref, v_ref, wo_ref, out_ref,
             part_ref, rs0_ref, rs1_ref, rs2_ref,
             rs_send_sems, rs_recv_sems, ag_send_sems, ag_recv_sems):
        my = lax.axis_index("i")

        q = jnp.dot(x_ref[0], wq_ref[...],
                    preferred_element_type=jnp.float32) * SCALE

        for h in range(HQ):
            kh4 = k_ref[0, :, h, :].reshape(16, 4, 64, DH)
            vh4 = v_ref[0, :, h, :].reshape(16, 4, 64, DH)
            for c in range(4):
                kg = kh4[:, c].reshape(16 * 64, DH)
                vg = vh4[:, c].reshape(16 * 64, DH)
                qc = q[c * 64:(c + 1) * 64, h * DH:(h + 1) * DH]
                s = lax.dot_general(qc, kg, (((1,), (1,)), ((), ())),
                                    preferred_element_type=jnp.float32)
                w = jnp.exp(s)
                part_ref[c * 64:(c + 1) * 64, D + h:D + h + 1] = jnp.sum(
                    w, axis=1, keepdims=True)
                part_ref[c * 64:(c + 1) * 64, h * DH:(h + 1) * DH] = jnp.dot(
                    w, vg, preferred_element_type=jnp.float32)
        part_ref[:, D + HQ:COLS] = jnp.zeros((SQ, COLS - D - HQ),
                                             jnp.float32)

        barrier_sem = pltpu.get_barrier_semaphore()
        for m in (1, 2, 4):
            pl.semaphore_signal(barrier_sem, inc=1,
                                device_id=(lax.bitwise_xor(my, m),),
                                device_id_type=pl.DeviceIdType.MESH)
        pl.semaphore_wait(barrier_sem, 3)

        base = my * 0
        rs_recvs = (rs0_ref, rs1_ref, rs2_ref)
        for j, m in enumerate((4, 2, 1)):
            sz = SQ >> (j + 1)
            bit = lax.bitwise_and(lax.shift_right_logical(my, 2 - j), 1)
            keep_off = pl.multiple_of(base + bit * sz, 32)
            send_off = pl.multiple_of(base + (1 - bit) * sz, 32)
            rdma = pltpu.make_async_remote_copy(
                src_ref=part_ref.at[pl.ds(send_off, sz), :],
                dst_ref=rs_recvs[j],
                send_sem=rs_send_sems.at[j],
                recv_sem=rs_recv_sems.at[j],
                device_id=(lax.bitwise_xor(my, m),),
                device_id_type=pl.DeviceIdType.MESH,
            )
            rdma.start()
            rdma.wait()
            part_ref[pl.ds(keep_off, sz), :] = (
                part_ref[pl.ds(keep_off, sz), :] + rs_recvs[j][...])
            base = keep_off
        base = pl.multiple_of(base, 32)

        mine = part_ref[pl.ds(base, OWN), :]
        ctx_cols = []
        for h in range(HQ):
            ctx_cols.append(mine[:, h * DH:(h + 1) * DH]
                            / mine[:, D + h:D + h + 1])
        ctx = jnp.concatenate(ctx_cols, axis=1)
        out_ref[0, pl.ds(base, OWN), :] = jnp.dot(
            ctx, wo_ref[...], preferred_element_type=jnp.float32)

        own_off = base
        for j, m in enumerate((1, 2, 4)):
            sz = OWN << j
            own_off = pl.multiple_of(own_off, 32)
            rdma = pltpu.make_async_remote_copy(
                src_ref=out_ref.at[0, pl.ds(own_off, sz), :],
                dst_ref=out_ref.at[0, pl.ds(own_off, sz), :],
                send_sem=ag_send_sems.at[j],
                recv_sem=ag_recv_sems.at[j],
                device_id=(lax.bitwise_xor(my, m),),
                device_id_type=pl.DeviceIdType.MESH,
            )
            rdma.start()
            rdma.wait()
            own_off = own_off & ~sz

    return pl.pallas_call(
        body,
        out_shape=jax.ShapeDtypeStruct((1, SQ, D), jnp.float32),
        in_specs=[pl.BlockSpec(memory_space=pltpu.VMEM)] * 5,
        out_specs=pl.BlockSpec(memory_space=pltpu.VMEM),
        scratch_shapes=[
            pltpu.VMEM((SQ, COLS), jnp.float32),
            pltpu.VMEM((SQ // 2, COLS), jnp.float32),
            pltpu.VMEM((SQ // 4, COLS), jnp.float32),
            pltpu.VMEM((SQ // 8, COLS), jnp.float32),
            pltpu.SemaphoreType.DMA((3,)),
            pltpu.SemaphoreType.DMA((3,)),
            pltpu.SemaphoreType.DMA((3,)),
            pltpu.SemaphoreType.DMA((3,)),
        ],
        compiler_params=pltpu.CompilerParams(
            collective_id=0, vmem_limit_bytes=100 * 1024 * 1024),
    )(x, Wq, K_ext, V_ext, Wo)
